# baseline (speedup 1.0000x reference)
import functools

import jax
import jax.numpy as jnp
from jax import lax
from jax.experimental import pallas as pl
from jax.experimental.pallas import tpu as pltpu
from jax.experimental.pallas import tpu_sc as plsc

_NC = 2
_NS = 16
_L = 16
_NW = _NC * _NS

_S1 = 16384
_D1 = 2048
_D2 = 512
_F = 128

_SRC_PER_W = _S1 // _NW
_DST_PER_W = _D1 // _NW


def _sc_gather_body(raw_hbm, nodes_hbm, d2s1_hbm, d2d1_hbm,
                    src1_out, dst1_out,
                    nodes_v, d2s1_v, idxs_v, rows_v,
                    d2d1_v, idxd_v, rowsd_v, sem):
    c = lax.axis_index("c")
    s = lax.axis_index("s")
    wid = s * _NC + c

    pltpu.sync_copy(nodes_hbm, nodes_v)

    base = wid * _SRC_PER_W
    pltpu.sync_copy(d2s1_hbm.at[pl.ds(base, _SRC_PER_W)], d2s1_v)
    for i in range(_SRC_PER_W // _L):
        j = d2s1_v[pl.ds(i * _L, _L)]
        composed = plsc.load_gather(nodes_v, [j])
        idxs_v[i // 8, pl.ds((i % 8) * _L, _L)] = composed
    copies = [
        pltpu.async_copy(raw_hbm.at[idxs_v.at[cc]],
                         rows_v.at[pl.ds(cc * 128, 128)], sem)
        for cc in range(_SRC_PER_W // 128)
    ]
    for cp in copies:
        cp.wait()
    pltpu.sync_copy(rows_v, src1_out.at[pl.ds(base, _SRC_PER_W)])

    based = wid * _DST_PER_W
    pltpu.sync_copy(d2d1_hbm.at[pl.ds(based, _DST_PER_W)], d2d1_v)
    for i in range(_DST_PER_W // _L):
        j = d2d1_v[pl.ds(i * _L, _L)]
        idxd_v[pl.ds(i * _L, _L)] = plsc.load_gather(nodes_v, [j])
    pltpu.async_copy(raw_hbm.at[idxd_v], rowsd_v, sem).wait()
    pltpu.sync_copy(rowsd_v, dst1_out.at[pl.ds(based, _DST_PER_W)])


def _sc_gather(raw_features, src_nodes, dstsrc2src1, dstsrc2dst1):
    mesh = plsc.VectorSubcoreMesh(core_axis_name="c", subcore_axis_name="s")
    k = pl.kernel(
        _sc_gather_body,
        out_type=(
            jax.ShapeDtypeStruct((_S1, _F), jnp.float32),
            jax.ShapeDtypeStruct((_D1, _F), jnp.float32),
        ),
        mesh=mesh,
        scratch_types=[
            pltpu.VMEM((_S1,), jnp.int32),
            pltpu.VMEM((_SRC_PER_W,), jnp.int32),
            pltpu.VMEM((_SRC_PER_W // 128, 128), jnp.int32),
            pltpu.VMEM((_SRC_PER_W, _F), jnp.float32),
            pltpu.VMEM((_DST_PER_W,), jnp.int32),
            pltpu.VMEM((_DST_PER_W,), jnp.int32),
            pltpu.VMEM((_DST_PER_W, _F), jnp.float32),
            pltpu.SemaphoreType.DMA,
        ],
        compiler_params=pltpu.CompilerParams(needs_layout_passes=False),
    )
    return k(raw_features, src_nodes, dstsrc2src1, dstsrc2dst1)


_MB = 128
_NM = _D1 // (2 * _MB)


def _tc_body(d2s2_ref, d2d2_ref, dif1a_ref, dif1b_ref, src1_ref,
             dst1a_ref, dst1b_ref, dif2_ref,
             wt_ref, wb_ref, W1_ref, b1_ref, W2_ref, b2_ref, W3_ref, b3_ref,
             out_ref, src1b_ref, h1_ref):
    m = pl.program_id(0)
    f32 = jnp.float32
    bf16 = jnp.bfloat16

    @pl.when(m == 0)
    def _init():
        src1b_ref[...] = src1_ref[...].astype(bf16)

    for half, (dif_ref, dst_ref) in enumerate(
            [(dif1a_ref, dst1a_ref), (dif1b_ref, dst1b_ref)]):
        agg1 = jnp.dot(dif_ref[...].astype(bf16), src1b_ref[...],
                       preferred_element_type=f32)
        h1_ref[pl.ds((2 * m + half) * _MB, _MB), :] = (
            jnp.dot(agg1, wt_ref[...], preferred_element_type=f32)
            + jnp.dot(dst_ref[...], wb_ref[...], preferred_element_type=f32))

    @pl.when(m == _NM - 1)
    def _tail():
        wt = wt_ref[...]
        wb = wb_ref[...]
        h1 = h1_ref[...].astype(bf16)
        oh_src = (lax.broadcasted_iota(jnp.int32, (_D1, _D1), 1)
                  == d2s2_ref[...]).astype(bf16)
        src2 = jnp.dot(oh_src, h1, preferred_element_type=f32)
        agg2 = jnp.dot(dif2_ref[...].astype(bf16), src2.astype(bf16),
                       preferred_element_type=f32)
        oh_dst = (lax.broadcasted_iota(jnp.int32, (_D2, _D1), 1)
                  == d2d2_ref[...]).astype(bf16)
        dst2 = jnp.dot(oh_dst, h1, preferred_element_type=f32)
        h2 = (jnp.dot(agg2, wt, preferred_element_type=f32)
              + jnp.dot(dst2, wb, preferred_element_type=f32))
        nrm = jnp.sqrt(jnp.maximum(
            jnp.sum(h2 * h2, axis=1, keepdims=True), 1e-12))
        emb = h2 / nrm
        h = jnp.maximum(
            jnp.dot(emb, W1_ref[...], preferred_element_type=f32)
            + b1_ref[...], 0.0)
        h = jnp.maximum(
            jnp.dot(h, W2_ref[...], preferred_element_type=f32)
            + b2_ref[...], 0.0)
        out_ref[...] = (jnp.dot(h, W3_ref[...], preferred_element_type=f32)
                        + b3_ref[...])


def _tc_compute(d2s2, d2d2, dif_mat1, src1, dst1, dif_mat2,
                wt, wb, W1, b1, W2, b2, W3, b3, interpret=False):
    whole = lambda shape: pl.BlockSpec(shape, lambda m: (0, 0))
    return pl.pallas_call(
        _tc_body,
        grid=(_NM,),
        in_specs=[
            whole((_D1, 1)),
            whole((_D2, 1)),
            pl.BlockSpec((_MB, _S1), lambda m: (2 * m, 0)),
            pl.BlockSpec((_MB, _S1), lambda m: (2 * m + 1, 0)),
            whole((_S1, _F)),
            pl.BlockSpec((_MB, _F), lambda m: (2 * m, 0)),
            pl.BlockSpec((_MB, _F), lambda m: (2 * m + 1, 0)),
            whole((_D2, _D1)),
            whole((_F, _F)),
            whole((_F, _F)),
            whole((_F, 64)),
            whole((1, 64)),
            whole((64, 64)),
            whole((1, 64)),
            whole((64, 8)),
            whole((1, 8)),
        ],
        out_specs=pl.BlockSpec((_D2, 8), lambda m: (0, 0)),
        out_shape=jax.ShapeDtypeStruct((_D2, 8), jnp.float32),
        scratch_shapes=[
            pltpu.VMEM((_S1, _F), jnp.bfloat16),
            pltpu.VMEM((_D1, _F), jnp.float32),
        ],
        compiler_params=pltpu.CompilerParams(
            dimension_semantics=("arbitrary",),
            vmem_limit_bytes=112 * 1024 * 1024),
        interpret=interpret,
    )(d2s2, d2d2, dif_mat1, dif_mat1, src1, dst1, dst1, dif_mat2,
      wt, wb, W1, b1, W2, b2, W3, b3)


@jax.jit
def kernel(raw_features, src_nodes, dstsrc2src1, dstsrc2dst1, dif_mat1,
           dstsrc2src2, dstsrc2dst2, dif_mat2, w_agg, W1, b1, W2, b2, W3, b3):
    src1, dst1 = _sc_gather(raw_features, src_nodes, dstsrc2src1, dstsrc2dst1)
    return _tc_compute(
        dstsrc2src2.reshape(_D1, 1), dstsrc2dst2.reshape(_D2, 1),
        dif_mat1, src1, dst1, dif_mat2,
        w_agg[:_F], w_agg[_F:], W1, b1.reshape(1, 64),
        W2, b2.reshape(1, 64), W3, b3.reshape(1, 8))


# speedup vs baseline: 1.0493x; 1.0493x over previous
import functools

import jax
import jax.numpy as jnp
from jax import lax
from jax.experimental import pallas as pl
from jax.experimental.pallas import tpu as pltpu
from jax.experimental.pallas import tpu_sc as plsc

_NC = 2
_NS = 16
_L = 16
_NW = _NC * _NS

_S1 = 16384
_D1 = 2048
_D2 = 512
_F = 128

_SRC_PER_W = _S1 // _NW
_DST_PER_W = _D1 // _NW


def _sc_gather_body(raw_hbm, nodes_hbm, d2s1_hbm, d2d1_hbm,
                    src1_out, dst1_out,
                    nodes_v, d2s1_v, idxs_v, rows_v,
                    d2d1_v, idxd_v, rowsd_v, sem):
    c = lax.axis_index("c")
    s = lax.axis_index("s")
    wid = s * _NC + c
    base = wid * _SRC_PER_W
    based = wid * _DST_PER_W

    stage = [
        pltpu.async_copy(nodes_hbm, nodes_v, sem),
        pltpu.async_copy(d2s1_hbm.at[pl.ds(base, _SRC_PER_W)], d2s1_v, sem),
        pltpu.async_copy(d2d1_hbm.at[pl.ds(based, _DST_PER_W)], d2d1_v, sem),
    ]
    for cp in stage:
        cp.wait()

    for i in range(_SRC_PER_W // _L):
        j = d2s1_v[pl.ds(i * _L, _L)]
        composed = plsc.load_gather(nodes_v, [j])
        idxs_v[i // 8, pl.ds((i % 8) * _L, _L)] = composed
    for i in range(_DST_PER_W // _L):
        j = d2d1_v[pl.ds(i * _L, _L)]
        idxd_v[pl.ds(i * _L, _L)] = plsc.load_gather(nodes_v, [j])

    copies = [
        pltpu.async_copy(raw_hbm.at[idxs_v.at[cc]],
                         rows_v.at[pl.ds(cc * 128, 128)], sem)
        for cc in range(_SRC_PER_W // 128)
    ]
    copies.append(pltpu.async_copy(raw_hbm.at[idxd_v], rowsd_v, sem))
    for cp in copies:
        cp.wait()

    outs = [
        pltpu.async_copy(rows_v, src1_out.at[pl.ds(base, _SRC_PER_W)], sem),
        pltpu.async_copy(rowsd_v, dst1_out.at[pl.ds(based, _DST_PER_W)], sem),
    ]
    for cp in outs:
        cp.wait()


def _sc_gather(raw_features, src_nodes, dstsrc2src1, dstsrc2dst1):
    mesh = plsc.VectorSubcoreMesh(core_axis_name="c", subcore_axis_name="s")
    k = pl.kernel(
        _sc_gather_body,
        out_type=(
            jax.ShapeDtypeStruct((_S1, _F), jnp.float32),
            jax.ShapeDtypeStruct((_D1, _F), jnp.float32),
        ),
        mesh=mesh,
        scratch_types=[
            pltpu.VMEM((_S1,), jnp.int32),
            pltpu.VMEM((_SRC_PER_W,), jnp.int32),
            pltpu.VMEM((_SRC_PER_W // 128, 128), jnp.int32),
            pltpu.VMEM((_SRC_PER_W, _F), jnp.float32),
            pltpu.VMEM((_DST_PER_W,), jnp.int32),
            pltpu.VMEM((_DST_PER_W,), jnp.int32),
            pltpu.VMEM((_DST_PER_W, _F), jnp.float32),
            pltpu.SemaphoreType.DMA,
        ],
        compiler_params=pltpu.CompilerParams(needs_layout_passes=False),
    )
    return k(raw_features, src_nodes, dstsrc2src1, dstsrc2dst1)


_MB = 128
_NM = _D1 // _MB


def _tc_body(d2s2_ref, d2d2_ref, dif1_ref, src1_ref, dst1_ref, dif2_ref,
             wt_ref, wb_ref, W1_ref, b1_ref, W2_ref, b2_ref, W3_ref, b3_ref,
             out_ref, src1b_ref, h1_ref):
    m = pl.program_id(0)
    f32 = jnp.float32
    bf16 = jnp.bfloat16

    @pl.when(m == 0)
    def _init():
        src1b_ref[...] = src1_ref[...].astype(bf16)

    agg1 = jnp.dot(dif1_ref[...].astype(bf16), src1b_ref[...],
                   preferred_element_type=f32)
    h1_ref[pl.ds(m * _MB, _MB), :] = (
        jnp.dot(agg1, wt_ref[...], preferred_element_type=f32)
        + jnp.dot(dst1_ref[...], wb_ref[...], preferred_element_type=f32))

    @pl.when(m == _NM - 1)
    def _tail():
        wt = wt_ref[...]
        wb = wb_ref[...]
        h1 = h1_ref[...].astype(bf16)
        oh_src = (lax.broadcasted_iota(jnp.int32, (_D1, _D1), 1)
                  == d2s2_ref[...]).astype(bf16)
        src2 = jnp.dot(oh_src, h1, preferred_element_type=f32)
        agg2 = jnp.dot(dif2_ref[...].astype(bf16), src2.astype(bf16),
                       preferred_element_type=f32)
        oh_dst = (lax.broadcasted_iota(jnp.int32, (_D2, _D1), 1)
                  == d2d2_ref[...]).astype(bf16)
        dst2 = jnp.dot(oh_dst, h1, preferred_element_type=f32)
        h2 = (jnp.dot(agg2, wt, preferred_element_type=f32)
              + jnp.dot(dst2, wb, preferred_element_type=f32))
        nrm = jnp.sqrt(jnp.maximum(
            jnp.sum(h2 * h2, axis=1, keepdims=True), 1e-12))
        emb = h2 / nrm
        h = jnp.maximum(
            jnp.dot(emb, W1_ref[...], preferred_element_type=f32)
            + b1_ref[...], 0.0)
        h = jnp.maximum(
            jnp.dot(h, W2_ref[...], preferred_element_type=f32)
            + b2_ref[...], 0.0)
        out_ref[...] = (jnp.dot(h, W3_ref[...], preferred_element_type=f32)
                        + b3_ref[...])


def _tc_compute(d2s2, d2d2, dif_mat1, src1, dst1, dif_mat2,
                wt, wb, W1, b1, W2, b2, W3, b3, interpret=False):
    whole = lambda shape: pl.BlockSpec(shape, lambda m: (0, 0))
    return pl.pallas_call(
        _tc_body,
        grid=(_NM,),
        in_specs=[
            whole((_D1, 1)),
            whole((_D2, 1)),
            pl.BlockSpec((_MB, _S1), lambda m: (m, 0)),
            whole((_S1, _F)),
            pl.BlockSpec((_MB, _F), lambda m: (m, 0)),
            whole((_D2, _D1)),
            whole((_F, _F)),
            whole((_F, _F)),
            whole((_F, 64)),
            whole((1, 64)),
            whole((64, 64)),
            whole((1, 64)),
            whole((64, 8)),
            whole((1, 8)),
        ],
        out_specs=pl.BlockSpec((_D2, 8), lambda m: (0, 0)),
        out_shape=jax.ShapeDtypeStruct((_D2, 8), jnp.float32),
        scratch_shapes=[
            pltpu.VMEM((_S1, _F), jnp.bfloat16),
            pltpu.VMEM((_D1, _F), jnp.float32),
        ],
        compiler_params=pltpu.CompilerParams(
            dimension_semantics=("arbitrary",),
            vmem_limit_bytes=112 * 1024 * 1024),
        interpret=interpret,
    )(d2s2, d2d2, dif_mat1, src1, dst1, dif_mat2,
      wt, wb, W1, b1, W2, b2, W3, b3)


@jax.jit
def kernel(raw_features, src_nodes, dstsrc2src1, dstsrc2dst1, dif_mat1,
           dstsrc2src2, dstsrc2dst2, dif_mat2, w_agg, W1, b1, W2, b2, W3, b3):
    src1, dst1 = _sc_gather(raw_features, src_nodes, dstsrc2src1, dstsrc2dst1)
    return _tc_compute(
        dstsrc2src2.reshape(_D1, 1), dstsrc2dst2.reshape(_D2, 1),
        dif_mat1, src1, dst1, dif_mat2,
        w_agg[:_F], w_agg[_F:], W1, b1.reshape(1, 64),
        W2, b2.reshape(1, 64), W3, b3.reshape(1, 8))
